# baseline (speedup 1.0000x reference)
import jax
from jax.experimental import pallas as pl


def _add_body(x_ref, pe_ref, out_ref):
    out_ref[...] = x_ref[...] + pe_ref[...]


def kernel(x, pe_table):
    batch, seq_len, d_model = x.shape
    pe = pe_table[:seq_len]
    x2 = x.reshape(batch * seq_len, d_model)
    out = pl.pallas_call(
        _add_body,
        grid=(batch,),
        in_specs=[
            pl.BlockSpec((seq_len, d_model), lambda b: (b, 0)),
            pl.BlockSpec((seq_len, d_model), lambda b: (0, 0)),
        ],
        out_specs=pl.BlockSpec((seq_len, d_model), lambda b: (b, 0)),
        out_shape=jax.ShapeDtypeStruct((batch * seq_len, d_model), x.dtype),
    )(x2, pe)
    return out.reshape(batch, seq_len, d_model)
